# Initial kernel scaffold; baseline (speedup 1.0000x reference)
import jax
import jax.numpy as jnp
from jax.experimental import pallas as pl


def kernel(x, edge_index, W1, att_src1, att_dst1, b1, W2, att_src2, att_dst2, b2):
    raise NotImplementedError("write your pallas kernel here")


# speedup vs baseline: 34.6479x; 34.6479x over previous
import functools

import jax
import jax.numpy as jnp
from jax import lax
from jax.experimental import pallas as pl
from jax.experimental.pallas import tpu as pltpu
from jax.experimental.pallas import tpu_sc as plsc

N = 10000
E = 320000
IN = 128
HID = 128
OUT = 64

EP = E + N
NT = 32
TPB = 10320
EPAD = NT * TPB
K = 80
NB = TPB // K
NACC = 10240
RPT = NACC // 16
RCH = 8


def _tc1_body(x_ref, w_ref, avs_ref, avd_ref, h_ref, as_ref, ad_ref):
    h = jnp.dot(x_ref[...], w_ref[...], preferred_element_type=jnp.float32)
    h_ref[...] = h
    as_ref[...] = jnp.dot(h, avs_ref[...], preferred_element_type=jnp.float32)
    ad_ref[...] = jnp.dot(h, avd_ref[...], preferred_element_type=jnp.float32)


_tc1 = pl.pallas_call(
    _tc1_body,
    out_shape=[
        jax.ShapeDtypeStruct((N, HID), jnp.float32),
        jax.ShapeDtypeStruct((N, 1), jnp.float32),
        jax.ShapeDtypeStruct((N, 1), jnp.float32),
    ],
)


def _tc2_body(acc_ref, es_ref, b1_ref, w2_ref, avs_ref, avd_ref,
              h2_ref, as_ref, ad_ref):
    es = jnp.sum(es_ref[...], axis=0)[:N]
    acc = acc_ref[0, :N, :] + acc_ref[1, :N, :]
    h1 = acc / (es[:, None] + 1e-16) + b1_ref[...]
    h1 = jnp.where(h1 > 0, h1, jnp.exp(jnp.minimum(h1, 0.0)) - 1.0)
    h2 = jnp.dot(h1, w2_ref[...], preferred_element_type=jnp.float32)
    h2_ref[...] = jnp.concatenate([h2, jnp.zeros_like(h2)], axis=1)
    as_ref[...] = jnp.dot(h2, avs_ref[...], preferred_element_type=jnp.float32)
    ad_ref[...] = jnp.dot(h2, avd_ref[...], preferred_element_type=jnp.float32)


_tc2 = pl.pallas_call(
    _tc2_body,
    out_shape=[
        jax.ShapeDtypeStruct((N, 2 * OUT), jnp.float32),
        jax.ShapeDtypeStruct((N, 1), jnp.float32),
        jax.ShapeDtypeStruct((N, 1), jnp.float32),
    ],
)


def _tc3_body(acc_ref, es_ref, b2_ref, out_ref):
    es = jnp.sum(es_ref[...], axis=0)[:N]
    acc = acc_ref[0, :N, :OUT] + acc_ref[1, :N, :OUT]
    out_ref[...] = acc / (es[:, None] + 1e-16) + b2_ref[...]


_tc3 = pl.pallas_call(
    _tc3_body,
    out_shape=jax.ShapeDtypeStruct((N, OUT), jnp.float32),
)


def _make_sc_edge(C):
    mesh = plsc.VectorSubcoreMesh(core_axis_name="c", subcore_axis_name="s")
    cpr = C // 16

    @functools.partial(
        pl.kernel,
        mesh=mesh,
        out_type=[
            jax.ShapeDtypeStruct((2, NACC, C), jnp.float32),
            jax.ShapeDtypeStruct((NT, NACC), jnp.float32),
        ],
        scratch_types=[
            pltpu.VMEM((N,), jnp.float32),
            pltpu.VMEM((N,), jnp.float32),
            pltpu.VMEM((2, K), jnp.int32),
            pltpu.VMEM((2, K), jnp.int32),
            pltpu.VMEM((NACC,), jnp.float32),
            pltpu.VMEM((K, C), jnp.float32),
            pltpu.VMEM_SHARED((NACC, C), jnp.float32),
            pltpu.SemaphoreType.DMA,
            pltpu.SemaphoreType.DMA,
        ],
        compiler_params=pltpu.CompilerParams(needs_layout_passes=False),
    )
    def sc_edge(h_hbm, asrc_hbm, adst_hbm, srcs_hbm, dsts_hbm,
                acc_hbm, esum_hbm,
                a_src_v, a_dst_v, srcb, dstb, esum_v, rows_v,
                acc_sh, sem, sem_idx):
        cid = lax.axis_index("c")
        sid = lax.axis_index("s")
        wid = sid * 2 + cid

        pltpu.sync_copy(asrc_hbm, a_src_v)
        pltpu.sync_copy(adst_hbm, a_dst_v)
        pltpu.sync_copy(srcs_hbm.at[wid, 0], srcb.at[0])
        pltpu.sync_copy(dsts_hbm.at[wid, 0], dstb.at[0])

        zeros16 = jnp.zeros((16,), jnp.float32)

        def _zrow(e, carry):
            for j in range(cpr):
                rows_v[e, pl.ds(j * 16, 16)] = zeros16
            return carry
        lax.fori_loop(0, K, _zrow, 0)

        def _zes(i, carry):
            esum_v[pl.ds(i * 16, 16)] = zeros16
            return carry
        lax.fori_loop(0, NACC // 16, _zes, 0)

        for i in range(RCH):
            pltpu.sync_copy(rows_v,
                            acc_sh.at[pl.ds(sid * RPT + i * K, K), :])
        plsc.subcore_barrier()

        ebase = wid * TPB
        lane = lax.iota(jnp.int32, 16)

        def _batch(b, carry):
            sl = lax.rem(b, 2)
            nsl = 1 - sl
            bn = jnp.minimum(b + 1, NB - 1)
            cps = pltpu.async_copy(srcs_hbm.at[wid, bn], srcb.at[nsl], sem_idx)
            cpd = pltpu.async_copy(dsts_hbm.at[wid, bn], dstb.at[nsl], sem_idx)
            cp = pltpu.async_copy(h_hbm.at[srcb.at[sl]], rows_v, sem)
            exs = []
            for l in range(K // 16):
                sidx = srcb[sl, pl.ds(l * 16, 16)]
                didx = dstb[sl, pl.ds(l * 16, 16)]
                a_s = plsc.load_gather(a_src_v, [sidx])
                a_d = plsc.load_gather(a_dst_v, [didx])
                al = a_s + a_d
                al = jnp.where(al >= 0, al, al * 0.2)
                ex = jnp.exp(al)
                eidx = ebase + b * K + (l * 16 + lane)
                ex = jnp.where(eidx < EP, ex, 0.0)
                exs.append(ex)
                plsc.addupdate_scatter(esum_v, [didx], ex)
            cp.wait()
            for l in range(K // 16):
                for e in range(16):
                    s = exs[l][e]
                    ee = l * 16 + e
                    for j in range(cpr):
                        rows_v[ee, pl.ds(j * 16, 16)] = (
                            rows_v[ee, pl.ds(j * 16, 16)] * s)
            pltpu.sync_copy(rows_v, acc_sh.at[dstb.at[sl]], add=True)
            cps.wait()
            cpd.wait()
            return carry

        lax.fori_loop(0, NB, _batch, 0)

        pltpu.sync_copy(esum_v, esum_hbm.at[wid])
        plsc.subcore_barrier()
        for i in range(RCH):
            r0 = sid * RPT + i * K
            pltpu.sync_copy(acc_sh.at[pl.ds(r0, K), :], rows_v)
            pltpu.sync_copy(rows_v, acc_hbm.at[cid, pl.ds(r0, K), :])

    return sc_edge


_sc_edge = _make_sc_edge(HID)


def kernel(x, edge_index, W1, att_src1, att_dst1, b1, W2, att_src2,
           att_dst2, b2):
    loop = jnp.arange(N, dtype=jnp.int32)
    pad = jnp.zeros((EPAD - EP,), jnp.int32)
    src = jnp.concatenate([edge_index[0], loop, pad]).reshape(NT, NB, K)
    dst = jnp.concatenate([edge_index[1], loop, pad]).reshape(NT, NB, K)

    h1, a1s, a1d = _tc1(x, W1, att_src1.reshape(HID, 1),
                        att_dst1.reshape(HID, 1))
    acc1, es1 = _sc_edge(h1, a1s.reshape(N), a1d.reshape(N), src, dst)
    h2, a2s, a2d = _tc2(acc1, es1, b1.reshape(1, HID), W2,
                        att_src2.reshape(OUT, 1), att_dst2.reshape(OUT, 1))
    acc2, es2 = _sc_edge(h2, a2s.reshape(N), a2d.reshape(N), src, dst)
    return _tc3(acc2, es2, b2.reshape(1, OUT))
